# baseline (speedup 1.0000x reference)
import functools

import jax
import jax.numpy as jnp
from jax import lax
from jax.experimental import pallas as pl
from jax.experimental.pallas import tpu as pltpu
from jax.experimental.pallas import tpu_sc as plsc

L = 16
NC = 2
NS = 16
NW = NC * NS

D = 12


@functools.lru_cache(maxsize=None)
def _make_lookup(rows: int, cols: int):
    assert rows % 8 == 0 and cols % (128 * NW) == 0
    n_jt = rows // 8
    cw = cols // NW
    CW = 256
    R = 4
    n_hw = cw // CW
    n_ch = n_jt * n_hw

    mesh = plsc.VectorSubcoreMesh(core_axis_name="c", subcore_axis_name="s")

    @functools.partial(
        pl.kernel,
        out_type=jax.ShapeDtypeStruct((D * rows, cols), jnp.float32),
        mesh=mesh,
        compiler_params=pltpu.CompilerParams(
            needs_layout_passes=False, use_tc_tiling_on_sc=True
        ),
        scratch_types=[
            pltpu.VMEM((48,), jnp.float32),
            pltpu.VMEM((16,), jnp.float32),
            pltpu.VMEM((24 * D,), jnp.float32),
        ] + [pltpu.VMEM((8, CW), jnp.int32) for _ in range(R)]
          + [pltpu.VMEM((D * 8, CW), jnp.float32) for _ in range(R)]
          + [
            pltpu.SemaphoreType.DMA,
            pltpu.SemaphoreType.DMA,
        ],
    )
    def lookup(cards_hbm, rank_hbm, suit_hbm, out_hbm,
               rank_v, suit_v, table_v, *rest):
        cbufs = rest[:R]
        obufs = rest[R:2 * R]
        csem, osem = rest[2 * R], rest[2 * R + 1]
        wid = lax.axis_index("s") * NC + lax.axis_index("c")
        pltpu.sync_copy(rank_hbm, rank_v)
        pltpu.sync_copy(suit_hbm, suit_v)

        lane = lax.iota(jnp.int32, L)
        for v in range(24 * D // L):
            pos = v * L + lane
            k = pos // 24
            c = pos - k * 24
            is_rank = k < 8
            ridx = jnp.where(is_rank, (c // 4) * 8 + k, 0)
            sidx = jnp.where(is_rank, 0, (c - (c // 4) * 4) * 4 + (k - 8))
            rv = plsc.load_gather(rank_v, [ridx])
            sv = plsc.load_gather(suit_v, [sidx])
            table_v[pl.ds(v * L, L)] = jnp.where(is_rank, rv, sv)

        ib0 = wid * cw

        def rowcol(ci):
            jt = ci // n_hw
            return jt * 8, ib0 + (ci - jt * n_hw) * CW

        def compute(b):
            cbuf = cbufs[b]
            obuf = obufs[b]
            for j in range(8):
                @plsc.parallel_loop(0, CW // L, unroll=8)
                def _(g):
                    c16 = cbuf[j, pl.ds(g * L, L)]
                    for k in range(D):
                        v = plsc.load_gather(
                            table_v.at[pl.ds(k * 24, 24)], [c16])
                        obuf[k * 8 + j, pl.ds(g * L, L)] = v

        def start_in(ci, b):
            r0, c0 = rowcol(ci)
            pltpu.async_copy(
                cards_hbm.at[pl.ds(r0, 8), pl.ds(c0, CW)], cbufs[b], csem,
            )

        def wait_in(b):
            pltpu.make_async_copy(
                cards_hbm.at[pl.ds(0, 8), pl.ds(ib0, CW)], cbufs[b], csem,
            ).wait()

        def start_out(ci, b):
            r0, c0 = rowcol(ci)
            for k in range(D):
                pltpu.async_copy(
                    obufs[b].at[pl.ds(k * 8, 8)],
                    out_hbm.at[pl.ds(k * rows + r0, 8), pl.ds(c0, CW)],
                    osem,
                )

        def wait_out(b):
            for k in range(D):
                pltpu.make_async_copy(
                    obufs[b].at[pl.ds(k * 8, 8)],
                    out_hbm.at[pl.ds(k * rows, 8), pl.ds(ib0, CW)],
                    osem,
                ).wait()

        for ci in range(R - 1):
            start_in(ci, ci)

        def body(pi, _):
            for b in range(R):
                ci = pi * R + b

                @pl.when(ci < n_ch)
                def _():
                    wait_in(b)

                    @pl.when(ci + R - 1 < n_ch)
                    def _():
                        start_in(ci + R - 1, (b + R - 1) % R)

                    @pl.when(ci >= R)
                    def _():
                        wait_out(b)

                    compute(b)
                    start_out(ci, b)
            return 0

        lax.fori_loop(0, (n_ch + R - 1) // R, body, 0)
        for b in range(R):
            wait_out(b)

    return lookup


def kernel(cards, rank_emb, suit_emb):
    b, s = cards.shape
    cards_t = cards.T.astype(jnp.int32)
    out2d = _make_lookup(s, b)(
        cards_t, rank_emb.reshape(-1), suit_emb.reshape(-1)
    )
    return out2d.reshape(D, s, b).transpose(2, 1, 0)


# speedup vs baseline: 1.1506x; 1.1506x over previous
import functools

import jax
import jax.numpy as jnp
from jax import lax
from jax.experimental import pallas as pl
from jax.experimental.pallas import tpu as pltpu
from jax.experimental.pallas import tpu_sc as plsc

L = 16
NC = 2
NS = 16
NW = NC * NS

D = 12


@functools.lru_cache(maxsize=None)
def _make_lookup(rows: int, cols: int):
    assert rows % 8 == 0 and cols % (128 * NW) == 0
    n_jt = rows // 8
    cw = cols // NW
    CW = 512
    R = 2
    n_hw = cw // CW
    n_ch = n_jt * n_hw

    mesh = plsc.VectorSubcoreMesh(core_axis_name="c", subcore_axis_name="s")

    @functools.partial(
        pl.kernel,
        out_type=jax.ShapeDtypeStruct((D * rows, cols), jnp.float32),
        mesh=mesh,
        compiler_params=pltpu.CompilerParams(
            needs_layout_passes=False, use_tc_tiling_on_sc=True
        ),
        scratch_types=[
            pltpu.VMEM((48,), jnp.float32),
            pltpu.VMEM((16,), jnp.float32),
            pltpu.VMEM((24 * D,), jnp.float32),
        ] + [pltpu.VMEM((8, CW), jnp.int32) for _ in range(R)]
          + [pltpu.VMEM((D * 8, CW), jnp.float32) for _ in range(R)]
          + [
            pltpu.SemaphoreType.DMA,
            pltpu.SemaphoreType.DMA,
        ],
    )
    def lookup(cards_hbm, rank_hbm, suit_hbm, out_hbm,
               rank_v, suit_v, table_v, *rest):
        cbufs = rest[:R]
        obufs = rest[R:2 * R]
        csem, osem = rest[2 * R], rest[2 * R + 1]
        wid = lax.axis_index("s") * NC + lax.axis_index("c")
        pltpu.sync_copy(rank_hbm, rank_v)
        pltpu.sync_copy(suit_hbm, suit_v)

        lane = lax.iota(jnp.int32, L)
        for v in range(24 * D // L):
            pos = v * L + lane
            k = pos // 24
            c = pos - k * 24
            is_rank = k < 8
            ridx = jnp.where(is_rank, (c // 4) * 8 + k, 0)
            sidx = jnp.where(is_rank, 0, (c - (c // 4) * 4) * 4 + (k - 8))
            rv = plsc.load_gather(rank_v, [ridx])
            sv = plsc.load_gather(suit_v, [sidx])
            table_v[pl.ds(v * L, L)] = jnp.where(is_rank, rv, sv)

        ib0 = wid * cw

        def rowcol(ci):
            jt = ci // n_hw
            return jt * 8, ib0 + (ci - jt * n_hw) * CW

        def compute(b):
            cbuf = cbufs[b]
            obuf = obufs[b]
            for j in range(8):
                @plsc.parallel_loop(0, CW // L, unroll=8)
                def _(g):
                    c16 = cbuf[j, pl.ds(g * L, L)]
                    for k in range(D):
                        v = plsc.load_gather(
                            table_v.at[pl.ds(k * 24, 24)], [c16])
                        obuf[k * 8 + j, pl.ds(g * L, L)] = v

        def start_in(ci, b):
            r0, c0 = rowcol(ci)
            pltpu.async_copy(
                cards_hbm.at[pl.ds(r0, 8), pl.ds(c0, CW)], cbufs[b], csem,
            )

        def wait_in(b):
            pltpu.make_async_copy(
                cards_hbm.at[pl.ds(0, 8), pl.ds(ib0, CW)], cbufs[b], csem,
            ).wait()

        def start_out(ci, b):
            r0, c0 = rowcol(ci)
            for k in range(D):
                pltpu.async_copy(
                    obufs[b].at[pl.ds(k * 8, 8)],
                    out_hbm.at[pl.ds(k * rows + r0, 8), pl.ds(c0, CW)],
                    osem,
                )

        def wait_out(b):
            for k in range(D):
                pltpu.make_async_copy(
                    obufs[b].at[pl.ds(k * 8, 8)],
                    out_hbm.at[pl.ds(k * rows, 8), pl.ds(ib0, CW)],
                    osem,
                ).wait()

        for ci in range(R - 1):
            start_in(ci, ci)

        def body(pi, _):
            for b in range(R):
                ci = pi * R + b

                @pl.when(ci < n_ch)
                def _():
                    wait_in(b)

                    @pl.when(ci + R - 1 < n_ch)
                    def _():
                        start_in(ci + R - 1, (b + R - 1) % R)

                    @pl.when(ci >= R)
                    def _():
                        wait_out(b)

                    compute(b)
                    start_out(ci, b)
            return 0

        lax.fori_loop(0, (n_ch + R - 1) // R, body, 0)
        for b in range(R):
            wait_out(b)

    return lookup


def kernel(cards, rank_emb, suit_emb):
    b, s = cards.shape
    cards_t = cards.T.astype(jnp.int32)
    out2d = _make_lookup(s, b)(
        cards_t, rank_emb.reshape(-1), suit_emb.reshape(-1)
    )
    return out2d.reshape(D, s, b).transpose(2, 1, 0)
